# baseline (speedup 1.0000x reference)
import functools

import jax
import jax.numpy as jnp
from jax import lax
from jax.experimental import pallas as pl
from jax.experimental.pallas import tpu as pltpu
from jax.experimental.pallas import tpu_sc as plsc

NC = 2
NS = 16
CH = 64


CH2 = 128


def _sc_scatter_body(hx0cv_ref, idrb_ref, idlb_ref,
                     srcv_ref, slcv_ref,
                     src_v, accr_v, accl_v, idxs_v,
                     idbr_v, idbl_v, sem1, sem2,
                     *, B, L, H):
    c = lax.axis_index("c")
    s = lax.axis_index("s")
    bpc = B // NC
    iota16 = lax.iota(jnp.int32, 16)
    zero16 = jnp.zeros((16,), jnp.float32)
    nch = L // CH2

    def build_idx2(base, k, slot):
        for m in range(CH2 // 16):
            idxs_v[slot, pl.ds(m * 16, 16)] = \
                (base + k * CH2 + m * 16 + iota16) * 16 + s

    def batch_body(j, carry):
        b = c * bpc + j
        base = b * L
        pltpu.sync_copy(idrb_ref.at[pl.ds(base, L)], idbr_v)
        pltpu.sync_copy(idlb_ref.at[pl.ds(base, L)], idbl_v)

        def zrow(g, cc):
            for q in range(8):
                r = g * 8 + q
                accr_v[r] = zero16
                accl_v[r] = zero16
            return cc

        lax.fori_loop(0, L // 8, zrow, 0)

        cps = []
        for k in range(nch):
            build_idx2(base, k, k)
            cps.append(pltpu.async_copy(
                hx0cv_ref.at[idxs_v.at[k]],
                src_v.at[pl.ds(k * CH2, CH2)], sem1))
        for cp in cps:
            cp.wait()

        def srow(g, cc):
            for q in range(32):
                r = g * 32 + q
                rv = src_v[r]
                plsc.addupdate_scatter(accr_v, [idbr_v[r], iota16], rv)
                plsc.addupdate_scatter(accl_v, [idbl_v[r], iota16], rv)
            return cc

        lax.fori_loop(0, L // 32, srow, 0)

        cps = []
        for k in range(nch):
            cps.append(pltpu.async_copy(
                accr_v.at[pl.ds(k * CH2, CH2)],
                srcv_ref.at[idxs_v.at[k]], sem1))
            cps.append(pltpu.async_copy(
                accl_v.at[pl.ds(k * CH2, CH2)],
                slcv_ref.at[idxs_v.at[k]], sem2))
        for cp in cps:
            cp.wait()
        return carry

    lax.fori_loop(0, bpc, batch_body, 0)


def _sc_gather_body(x_ref, hx0_ref, idd_ref, idr_ref, idl_ref,
                    xg_ref, hr_ref, hl_ref,
                    fd_v, fr_v, fl_v, xg_v, hr_v, hl_v,
                    sem1, sem2, sem3, *, B, L, H):
    c = lax.axis_index("c")
    s = lax.axis_index("s")
    wid = s * NC + c
    nrows = (B * L) // (NC * NS)
    gstart = wid * nrows

    def gchunk(g, carry):
        rs = gstart + g * CH
        bbase = (rs // L) * L
        pltpu.sync_copy(idd_ref.at[pl.ds(rs, CH)], fd_v)
        pltpu.sync_copy(idr_ref.at[pl.ds(rs, CH)], fr_v)
        pltpu.sync_copy(idl_ref.at[pl.ds(rs, CH)], fl_v)
        for k in range(CH // 16):
            sl_ = pl.ds(k * 16, 16)
            fd_v[sl_] = fd_v[sl_] + bbase
            fr_v[sl_] = fr_v[sl_] + bbase
            fl_v[sl_] = fl_v[sl_] + bbase
        cp1 = pltpu.async_copy(x_ref.at[fd_v], xg_v, sem1)
        cp2 = pltpu.async_copy(hx0_ref.at[fr_v], hr_v, sem2)
        cp3 = pltpu.async_copy(hx0_ref.at[fl_v], hl_v, sem3)
        cp1.wait()
        cp2.wait()
        cp3.wait()
        pltpu.sync_copy(xg_v, xg_ref.at[pl.ds(rs, CH)])
        pltpu.sync_copy(hr_v, hr_ref.at[pl.ds(rs, CH)])
        pltpu.sync_copy(hl_v, hl_ref.at[pl.ds(rs, CH)])
        return carry

    lax.fori_loop(0, nrows // CH, gchunk, 0)


def _sc_scatter_fc_body(fccv_ref, iddb_ref, cscv_ref,
                        src_v, acc_v, idxs_v, idbd_v, sem1, *, B, L, H):
    c = lax.axis_index("c")
    s = lax.axis_index("s")
    bpc = B // NC
    iota16 = lax.iota(jnp.int32, 16)
    zero16 = jnp.zeros((16,), jnp.float32)
    nch = L // CH2

    def build_idx2(base, k, slot):
        for m in range(CH2 // 16):
            idxs_v[slot, pl.ds(m * 16, 16)] = \
                (base + k * CH2 + m * 16 + iota16) * 16 + s

    def batch_body(j, carry):
        b = c * bpc + j
        base = b * L
        pltpu.sync_copy(iddb_ref.at[pl.ds(base, L)], idbd_v)

        def zrow(g, cc):
            for q in range(8):
                acc_v[g * 8 + q] = zero16
            return cc

        lax.fori_loop(0, L // 8, zrow, 0)

        cps = []
        for k in range(nch):
            build_idx2(base, k, k)
            cps.append(pltpu.async_copy(
                fccv_ref.at[idxs_v.at[k]],
                src_v.at[pl.ds(k * CH2, CH2)], sem1))
        for cp in cps:
            cp.wait()

        def srow(g, cc):
            for q in range(32):
                r = g * 32 + q
                plsc.addupdate_scatter(acc_v, [idbd_v[r], iota16],
                                       src_v[r])
            return cc

        lax.fori_loop(0, L // 32, srow, 0)

        cps = []
        for k in range(nch):
            cps.append(pltpu.async_copy(
                acc_v.at[pl.ds(k * CH2, CH2)],
                cscv_ref.at[idxs_v.at[k]], sem1))
        for cp in cps:
            cp.wait()
        return carry

    lax.fori_loop(0, bpc, batch_body, 0)


def _tc_prep_body(iddc_ref, idrc_ref, idlc_ref, od_ref, or_ref, ol_ref,
                  *, L):
    od_ref[0] = jnp.broadcast_to(iddc_ref[0], (L, 16))
    or_ref[0] = jnp.broadcast_to(idrc_ref[0], (L, 16))
    ol_ref[0] = jnp.broadcast_to(idlc_ref[0], (L, 16))


def _tc_dense_body(x_ref, idrr_ref, idlr_ref, wiou_ref,
                   xw_ref, cntr_ref, cntl_ref, *, L, H):
    f32 = jnp.float32
    xw_ref[0] = jnp.dot(x_ref[0], wiou_ref[...], preferred_element_type=f32)
    row = lax.broadcasted_iota(jnp.int32, (L, L), 0)
    ones = jnp.ones((L, 1), f32)
    cntr_ref[0] = jnp.dot((row == idrr_ref[0]).astype(f32), ones,
                          preferred_element_type=f32)
    cntl_ref[0] = jnp.dot((row == idlr_ref[0]).astype(f32), ones,
                          preferred_element_type=f32)


def _tc_gates_body(xw_ref, sr_ref, sl_ref, xg_ref, hr_ref, hl_ref, hx1_ref,
                   cntr_ref, cntl_ref, w0_ref, w1_ref, wfx_ref,
                   w01_ref, w23_ref, b0_ref, b1_ref, bf_ref,
                   fc_ref, iu_ref, o_ref, *, L, H):
    f32 = jnp.float32
    xw = xw_ref[0]
    s = jnp.dot(sr_ref[0], w0_ref[...], preferred_element_type=f32)
    s = s + jnp.dot(sl_ref[0], w1_ref[...], preferred_element_type=f32)
    s = s + cntr_ref[0] * b0_ref[...] + cntl_ref[0] * b1_ref[...]
    i = jax.nn.sigmoid(xw[:, :H] + s)
    o = jax.nn.sigmoid(xw[:, H:2 * H])
    u = jnp.tanh(xw[:, 2 * H:])
    fpre = jnp.dot(xg_ref[0], wfx_ref[...], preferred_element_type=f32)
    fpre = fpre + jnp.dot(hr_ref[0], w01_ref[...], preferred_element_type=f32)
    fpre = fpre + jnp.dot(hl_ref[0], w23_ref[...], preferred_element_type=f32)
    f = jax.nn.sigmoid(fpre + bf_ref[...])
    fc_ref[0] = f * hx1_ref[0]
    iu_ref[0] = i * u
    o_ref[0] = o


def _tc_combine_body(iu_ref, o_ref, cs_ref, h_ref, c_ref):
    c = iu_ref[0] + cs_ref[0]
    h_ref[0] = o_ref[0] * jnp.tanh(c)
    c_ref[0] = c


def _stage_b_body(hflat_ref, cflat_ref, hx0_ref, hx1_ref, iddc_ref,
                  hout_ref, cout_ref, bufh, bufc, semh, semc, base_ref,
                  *, L, H, BL):
    f32 = jnp.float32
    W = L + 16
    b = pl.program_id(0)

    @pl.when(b == 0)
    def _():
        base_ref[0] = 0

    base = base_ref[0]
    base8 = jnp.minimum((base // 8) * 8, BL - W)
    base8 = pl.multiple_of(base8, 8)
    off = base - base8
    cp1 = pltpu.make_async_copy(hflat_ref.at[pl.ds(base8, W)], bufh, semh)
    cp2 = pltpu.make_async_copy(cflat_ref.at[pl.ds(base8, W)], bufc, semc)
    cp1.start()
    cp2.start()
    tid_d = iddc_ref[0]
    mask = tid_d != 0
    mf = mask.astype(f32)
    row = lax.broadcasted_iota(jnp.int32, (L, L), 0)
    col = lax.broadcasted_iota(jnp.int32, (L, L), 1)
    tri = (col <= row).astype(f32)
    rank_inc = jnp.dot(tri, mf, preferred_element_type=f32)
    rank = rank_inc.astype(jnp.int32) - 1 + off
    colw = lax.broadcasted_iota(jnp.int32, (L, W), 1)
    p = ((rank == colw) & mask).astype(f32)
    cp1.wait()
    cp2.wait()
    gh = jnp.dot(p, bufh[...], preferred_element_type=f32)
    gc = jnp.dot(p, bufc[...], preferred_element_type=f32)
    hout_ref[0] = jnp.where(mask, gh, hx0_ref[0])
    cout_ref[0] = jnp.where(mask, gc, hx1_ref[0])
    base_ref[0] = base + jnp.sum(mask.astype(jnp.int32))


def kernel(x, hx_0, hx_1, tree_ids_d, tree_ids_dr, tree_ids_dl,
           W_ioux, W_iouh0, b_iouh0, W_iouh1, b_iouh1, W_fx,
           W_fh0, b_fh0, W_fh1, b_fh1, W_fh2, b_fh2, W_fh3, b_fh3):
    B, L, E = x.shape
    H = W_fx.shape[1]
    BL = B * L
    f32 = jnp.float32

    x2 = x.reshape(BL, E)
    hx0_2 = hx_0.reshape(BL, H)
    idd_f = tree_ids_d.reshape(BL)
    idr_f = tree_ids_dr.reshape(BL)
    idl_f = tree_ids_dl.reshape(BL)

    mesh = plsc.VectorSubcoreMesh(core_axis_name="c", subcore_axis_name="s",
                                  num_cores=NC, num_subcores=NS)
    rowsd = jax.ShapeDtypeStruct((BL, H), f32)
    cvsd = jax.ShapeDtypeStruct((BL * 16, 16), f32)
    hx0_cv = hx0_2.reshape(BL * 16, 16)

    idd_c = tree_ids_d.reshape(B, L, 1)
    idr_c = tree_ids_dr.reshape(B, L, 1)
    idl_c = tree_ids_dl.reshape(B, L, 1)
    colid_spec = pl.BlockSpec((1, L, 1), lambda b: (b, 0, 0))
    bc_spec = pl.BlockSpec((1, L, 16), lambda b: (b, 0, 0))
    idd_bc, idr_bc, idl_bc = pl.pallas_call(
        functools.partial(_tc_prep_body, L=L),
        grid=(B,),
        in_specs=[colid_spec, colid_spec, colid_spec],
        out_specs=[bc_spec, bc_spec, bc_spec],
        out_shape=[jax.ShapeDtypeStruct((B, L, 16), jnp.int32)] * 3,
        compiler_params=pltpu.CompilerParams(
            dimension_semantics=("arbitrary",)),
    )(idd_c, idr_c, idl_c)

    sc_params = pltpu.CompilerParams(needs_layout_passes=False,
                                     use_tc_tiling_on_sc=False)
    sc1a = functools.partial(
        pl.kernel,
        out_type=[cvsd, cvsd],
        mesh=mesh,
        scratch_types=[
            pltpu.VMEM((L, 16), f32),
            pltpu.VMEM((L, 16), f32),
            pltpu.VMEM((L, 16), f32),
            pltpu.VMEM((8, CH2), jnp.int32),
            pltpu.VMEM((L, 16), jnp.int32),
            pltpu.VMEM((L, 16), jnp.int32),
            pltpu.SemaphoreType.DMA,
            pltpu.SemaphoreType.DMA,
        ],
        compiler_params=sc_params,
    )(functools.partial(_sc_scatter_body, B=B, L=L, H=H))
    sr, sl = sc1a(hx0_cv, idr_bc.reshape(BL, 16), idl_bc.reshape(BL, 16))

    sc1b = functools.partial(
        pl.kernel,
        out_type=[rowsd, rowsd, rowsd],
        mesh=mesh,
        scratch_types=[
            pltpu.VMEM((CH,), jnp.int32),
            pltpu.VMEM((CH,), jnp.int32),
            pltpu.VMEM((CH,), jnp.int32),
            pltpu.VMEM((CH, H), f32),
            pltpu.VMEM((CH, H), f32),
            pltpu.VMEM((CH, H), f32),
            pltpu.SemaphoreType.DMA,
            pltpu.SemaphoreType.DMA,
            pltpu.SemaphoreType.DMA,
        ],
        compiler_params=sc_params,
    )(functools.partial(_sc_gather_body, B=B, L=L, H=H))
    xg, hr, hl = sc1b(x2, hx0_2, idd_f, idr_f, idl_f)

    wh_spec = pl.BlockSpec((H, H), lambda b: (0, 0))
    bias_spec = pl.BlockSpec((1, H), lambda b: (0, 0))
    bh_spec = pl.BlockSpec((1, L, H), lambda b: (b, 0, 0))
    row_spec = pl.BlockSpec((1, 1, L), lambda b: (b, 0, 0))
    cnt_spec = pl.BlockSpec((1, L, 1), lambda b: (b, 0, 0))
    idr_r = tree_ids_dr.reshape(B, 1, L)
    idl_r = tree_ids_dl.reshape(B, 1, L)
    xw, cnt_r, cnt_l = pl.pallas_call(
        functools.partial(_tc_dense_body, L=L, H=H),
        grid=(B,),
        in_specs=[
            pl.BlockSpec((1, L, E), lambda b: (b, 0, 0)),
            row_spec, row_spec,
            pl.BlockSpec((E, 3 * H), lambda b: (0, 0)),
        ],
        out_specs=[
            pl.BlockSpec((1, L, 3 * H), lambda b: (b, 0, 0)),
            cnt_spec, cnt_spec,
        ],
        out_shape=[
            jax.ShapeDtypeStruct((B, L, 3 * H), f32),
            jax.ShapeDtypeStruct((B, L, 1), f32),
            jax.ShapeDtypeStruct((B, L, 1), f32),
        ],
        compiler_params=pltpu.CompilerParams(
            dimension_semantics=("arbitrary",)),
    )(x, idr_r, idl_r, W_ioux)

    fc, iu, o = pl.pallas_call(
        functools.partial(_tc_gates_body, L=L, H=H),
        grid=(B,),
        in_specs=[
            pl.BlockSpec((1, L, 3 * H), lambda b: (b, 0, 0)),
            bh_spec, bh_spec, bh_spec, bh_spec, bh_spec, bh_spec,
            cnt_spec, cnt_spec,
            wh_spec, wh_spec,
            pl.BlockSpec((E, H), lambda b: (0, 0)),
            wh_spec, wh_spec,
            bias_spec, bias_spec, bias_spec,
        ],
        out_specs=[bh_spec, bh_spec, bh_spec],
        out_shape=[jax.ShapeDtypeStruct((B, L, H), f32)] * 3,
        compiler_params=pltpu.CompilerParams(
            dimension_semantics=("arbitrary",)),
    )(xw, sr.reshape(B, L, H), sl.reshape(B, L, H), xg.reshape(B, L, H),
      hr.reshape(B, L, H), hl.reshape(B, L, H), hx_1, cnt_r, cnt_l,
      W_iouh0[:, :H], W_iouh1[:, :H], W_fx,
      W_fh0 + W_fh1, W_fh2 + W_fh3,
      b_iouh0[:H].reshape(1, H), b_iouh1[:H].reshape(1, H),
      (b_fh0 + b_fh1 + b_fh2 + b_fh3).reshape(1, H))

    sc2 = functools.partial(
        pl.kernel,
        out_type=[cvsd],
        mesh=mesh,
        scratch_types=[
            pltpu.VMEM((L, 16), f32),
            pltpu.VMEM((L, 16), f32),
            pltpu.VMEM((8, CH2), jnp.int32),
            pltpu.VMEM((L, 16), jnp.int32),
            pltpu.SemaphoreType.DMA,
        ],
        compiler_params=pltpu.CompilerParams(needs_layout_passes=False, use_tc_tiling_on_sc=False),
    )(functools.partial(_sc_scatter_fc_body, B=B, L=L, H=H))
    (cs,) = sc2(fc.reshape(BL * 16, 16), idd_bc.reshape(BL, 16))

    h_full, c_full = pl.pallas_call(
        _tc_combine_body,
        grid=(B,),
        in_specs=[bh_spec, bh_spec, bh_spec],
        out_specs=[bh_spec, bh_spec],
        out_shape=[jax.ShapeDtypeStruct((B, L, H), f32)] * 2,
        compiler_params=pltpu.CompilerParams(
            dimension_semantics=("arbitrary",)),
    )(iu, o, cs.reshape(B, L, H))

    idd_c = tree_ids_d.reshape(B, L, 1)
    col_spec = pl.BlockSpec((1, L, 1), lambda b: (b, 0, 0))
    h_out, c_out = pl.pallas_call(
        functools.partial(_stage_b_body, L=L, H=H, BL=BL),
        grid=(B,),
        in_specs=[
            pl.BlockSpec(memory_space=pl.ANY),
            pl.BlockSpec(memory_space=pl.ANY),
            bh_spec, bh_spec,
            col_spec,
        ],
        out_specs=[bh_spec, bh_spec],
        out_shape=[jax.ShapeDtypeStruct((B, L, H), f32)] * 2,
        scratch_shapes=[
            pltpu.VMEM((L + 16, H), f32),
            pltpu.VMEM((L + 16, H), f32),
            pltpu.SemaphoreType.DMA,
            pltpu.SemaphoreType.DMA,
            pltpu.SMEM((1,), jnp.int32),
        ],
        compiler_params=pltpu.CompilerParams(
            dimension_semantics=("arbitrary",)),
    )(h_full.reshape(BL, H), c_full.reshape(BL, H), hx_0, hx_1, idd_c)

    return (h_out, c_out)


# speedup vs baseline: 1.0168x; 1.0168x over previous
import functools

import jax
import jax.numpy as jnp
from jax import lax
from jax.experimental import pallas as pl
from jax.experimental.pallas import tpu as pltpu
from jax.experimental.pallas import tpu_sc as plsc

NC = 2
NS = 16
CH = 64


CH2 = 128


def _sc_scatter_body(hx0cv_ref, idrb_ref, idlb_ref,
                     srcv_ref, slcv_ref,
                     src_v, accr_v, accl_v, idxs_v,
                     idbr_v, idbl_v, sem1, sem2,
                     *, B, L, H):
    c = lax.axis_index("c")
    s = lax.axis_index("s")
    bpc = B // NC
    iota16 = lax.iota(jnp.int32, 16)
    zero16 = jnp.zeros((16,), jnp.float32)
    nch = L // CH2

    def build_idx2(base, k, slot):
        for m in range(CH2 // 16):
            idxs_v[slot, pl.ds(m * 16, 16)] = \
                (base + k * CH2 + m * 16 + iota16) * 16 + s

    def batch_body(j, carry):
        b = c * bpc + j
        base = b * L
        pltpu.sync_copy(idrb_ref.at[pl.ds(base, L)], idbr_v)
        pltpu.sync_copy(idlb_ref.at[pl.ds(base, L)], idbl_v)

        def zrow(g, cc):
            for q in range(8):
                r = g * 8 + q
                accr_v[r] = zero16
                accl_v[r] = zero16
            return cc

        lax.fori_loop(0, L // 8, zrow, 0)

        cps = []
        for k in range(nch):
            build_idx2(base, k, k)
            cps.append(pltpu.async_copy(
                hx0cv_ref.at[idxs_v.at[k]],
                src_v.at[pl.ds(k * CH2, CH2)], sem1))
        for cp in cps:
            cp.wait()

        def srow(g, cc):
            for q in range(32):
                r = g * 32 + q
                rv = src_v[r]
                plsc.addupdate_scatter(accr_v, [idbr_v[r], iota16], rv)
                plsc.addupdate_scatter(accl_v, [idbl_v[r], iota16], rv)
            return cc

        lax.fori_loop(0, L // 32, srow, 0)

        cps = []
        for k in range(nch):
            cps.append(pltpu.async_copy(
                accr_v.at[pl.ds(k * CH2, CH2)],
                srcv_ref.at[idxs_v.at[k]], sem1))
            cps.append(pltpu.async_copy(
                accl_v.at[pl.ds(k * CH2, CH2)],
                slcv_ref.at[idxs_v.at[k]], sem2))
        for cp in cps:
            cp.wait()
        return carry

    lax.fori_loop(0, bpc, batch_body, 0)


def _sc_gather_body(x_ref, hx0_ref, idd_ref, idr_ref, idl_ref,
                    xg_ref, hr_ref, hl_ref,
                    fd_v, fr_v, fl_v, xg_v, hr_v, hl_v,
                    sem1, sem2, sem3, *, B, L, H):
    c = lax.axis_index("c")
    s = lax.axis_index("s")
    wid = s * NC + c
    nrows = (B * L) // (NC * NS)
    gstart = wid * nrows

    def gchunk(g, carry):
        rs = gstart + g * CH
        bbase = (rs // L) * L
        pltpu.sync_copy(idd_ref.at[pl.ds(rs, CH)], fd_v)
        pltpu.sync_copy(idr_ref.at[pl.ds(rs, CH)], fr_v)
        pltpu.sync_copy(idl_ref.at[pl.ds(rs, CH)], fl_v)
        for k in range(CH // 16):
            sl_ = pl.ds(k * 16, 16)
            fd_v[sl_] = fd_v[sl_] + bbase
            fr_v[sl_] = fr_v[sl_] + bbase
            fl_v[sl_] = fl_v[sl_] + bbase
        cp1 = pltpu.async_copy(x_ref.at[fd_v], xg_v, sem1)
        cp2 = pltpu.async_copy(hx0_ref.at[fr_v], hr_v, sem2)
        cp3 = pltpu.async_copy(hx0_ref.at[fl_v], hl_v, sem3)
        cp1.wait()
        cp2.wait()
        cp3.wait()
        pltpu.sync_copy(xg_v, xg_ref.at[pl.ds(rs, CH)])
        pltpu.sync_copy(hr_v, hr_ref.at[pl.ds(rs, CH)])
        pltpu.sync_copy(hl_v, hl_ref.at[pl.ds(rs, CH)])
        return carry

    lax.fori_loop(0, nrows // CH, gchunk, 0)


def _sc_scatter_fc_body(fccv_ref, iddb_ref, cscv_ref,
                        src_v, acc_v, idxs_v, idbd_v, sem1, *, B, L, H):
    c = lax.axis_index("c")
    s = lax.axis_index("s")
    bpc = B // NC
    iota16 = lax.iota(jnp.int32, 16)
    zero16 = jnp.zeros((16,), jnp.float32)
    nch = L // CH2

    def build_idx2(base, k, slot):
        for m in range(CH2 // 16):
            idxs_v[slot, pl.ds(m * 16, 16)] = \
                (base + k * CH2 + m * 16 + iota16) * 16 + s

    def batch_body(j, carry):
        b = c * bpc + j
        base = b * L
        pltpu.sync_copy(iddb_ref.at[pl.ds(base, L)], idbd_v)

        def zrow(g, cc):
            for q in range(8):
                acc_v[g * 8 + q] = zero16
            return cc

        lax.fori_loop(0, L // 8, zrow, 0)

        cps = []
        for k in range(nch):
            build_idx2(base, k, k)
            cps.append(pltpu.async_copy(
                fccv_ref.at[idxs_v.at[k]],
                src_v.at[pl.ds(k * CH2, CH2)], sem1))
        for cp in cps:
            cp.wait()

        def srow(g, cc):
            for q in range(32):
                r = g * 32 + q
                plsc.addupdate_scatter(acc_v, [idbd_v[r], iota16],
                                       src_v[r])
            return cc

        lax.fori_loop(0, L // 32, srow, 0)

        cps = []
        for k in range(nch):
            cps.append(pltpu.async_copy(
                acc_v.at[pl.ds(k * CH2, CH2)],
                cscv_ref.at[idxs_v.at[k]], sem1))
        for cp in cps:
            cp.wait()
        return carry

    lax.fori_loop(0, bpc, batch_body, 0)


def _tc_prep_body(iddc_ref, idrc_ref, idlc_ref, od_ref, or_ref, ol_ref,
                  *, L):
    od_ref[0] = jnp.broadcast_to(iddc_ref[0], (L, 16))
    or_ref[0] = jnp.broadcast_to(idrc_ref[0], (L, 16))
    ol_ref[0] = jnp.broadcast_to(idlc_ref[0], (L, 16))


def _tc_gates_body(x_ref, sr_ref, sl_ref, xg_ref, hr_ref, hl_ref, hx1_ref,
                   idrr_ref, idlr_ref, wiou_ref, w0_ref, w1_ref, wfx_ref,
                   w01_ref, w23_ref, b0_ref, b1_ref, bf_ref,
                   fc_ref, iu_ref, o_ref, *, L, H):
    f32 = jnp.float32
    xw = jnp.dot(x_ref[0], wiou_ref[...], preferred_element_type=f32)
    s = jnp.dot(sr_ref[0], w0_ref[...], preferred_element_type=f32)
    s = s + jnp.dot(sl_ref[0], w1_ref[...], preferred_element_type=f32)
    row = lax.broadcasted_iota(jnp.int32, (L, L), 0)
    ones = jnp.ones((L, 1), f32)
    cnt_r = jnp.dot((row == idrr_ref[0]).astype(f32), ones,
                    preferred_element_type=f32)
    cnt_l = jnp.dot((row == idlr_ref[0]).astype(f32), ones,
                    preferred_element_type=f32)
    s = s + cnt_r * b0_ref[...] + cnt_l * b1_ref[...]
    i = jax.nn.sigmoid(xw[:, :H] + s)
    o = jax.nn.sigmoid(xw[:, H:2 * H])
    u = jnp.tanh(xw[:, 2 * H:])
    fpre = jnp.dot(xg_ref[0], wfx_ref[...], preferred_element_type=f32)
    fpre = fpre + jnp.dot(hr_ref[0], w01_ref[...], preferred_element_type=f32)
    fpre = fpre + jnp.dot(hl_ref[0], w23_ref[...], preferred_element_type=f32)
    f = jax.nn.sigmoid(fpre + bf_ref[...])
    fc_ref[0] = f * hx1_ref[0]
    iu_ref[0] = i * u
    o_ref[0] = o


def _tc_combine_body(iu_ref, o_ref, cs_ref, h_ref, c_ref):
    c = iu_ref[0] + cs_ref[0]
    h_ref[0] = o_ref[0] * jnp.tanh(c)
    c_ref[0] = c


def _stage_b_body(hflat_ref, cflat_ref, hx0_ref, hx1_ref, iddc_ref,
                  hout_ref, cout_ref, bufh, bufc, semh, semc, base_ref,
                  *, L, H, BL):
    f32 = jnp.float32
    W = L + 16
    b = pl.program_id(0)

    @pl.when(b == 0)
    def _():
        base_ref[0] = 0

    base = base_ref[0]
    base8 = jnp.minimum((base // 8) * 8, BL - W)
    base8 = pl.multiple_of(base8, 8)
    off = base - base8
    cp1 = pltpu.make_async_copy(hflat_ref.at[pl.ds(base8, W)], bufh, semh)
    cp2 = pltpu.make_async_copy(cflat_ref.at[pl.ds(base8, W)], bufc, semc)
    cp1.start()
    cp2.start()
    tid_d = iddc_ref[0]
    mask = tid_d != 0
    mf = mask.astype(f32)
    row = lax.broadcasted_iota(jnp.int32, (L, L), 0)
    col = lax.broadcasted_iota(jnp.int32, (L, L), 1)
    tri = (col <= row).astype(f32)
    rank_inc = jnp.dot(tri, mf, preferred_element_type=f32)
    rank = rank_inc.astype(jnp.int32) - 1 + off
    colw = lax.broadcasted_iota(jnp.int32, (L, W), 1)
    p = ((rank == colw) & mask).astype(f32)
    cp1.wait()
    cp2.wait()
    gh = jnp.dot(p, bufh[...], preferred_element_type=f32)
    gc = jnp.dot(p, bufc[...], preferred_element_type=f32)
    hout_ref[0] = jnp.where(mask, gh, hx0_ref[0])
    cout_ref[0] = jnp.where(mask, gc, hx1_ref[0])
    base_ref[0] = base + jnp.sum(mask.astype(jnp.int32))


def kernel(x, hx_0, hx_1, tree_ids_d, tree_ids_dr, tree_ids_dl,
           W_ioux, W_iouh0, b_iouh0, W_iouh1, b_iouh1, W_fx,
           W_fh0, b_fh0, W_fh1, b_fh1, W_fh2, b_fh2, W_fh3, b_fh3):
    B, L, E = x.shape
    H = W_fx.shape[1]
    BL = B * L
    f32 = jnp.float32

    x2 = x.reshape(BL, E)
    hx0_2 = hx_0.reshape(BL, H)
    idd_f = tree_ids_d.reshape(BL)
    idr_f = tree_ids_dr.reshape(BL)
    idl_f = tree_ids_dl.reshape(BL)

    mesh = plsc.VectorSubcoreMesh(core_axis_name="c", subcore_axis_name="s",
                                  num_cores=NC, num_subcores=NS)
    rowsd = jax.ShapeDtypeStruct((BL, H), f32)
    cvsd = jax.ShapeDtypeStruct((BL * 16, 16), f32)
    hx0_cv = hx0_2.reshape(BL * 16, 16)

    idd_c = tree_ids_d.reshape(B, L, 1)
    idr_c = tree_ids_dr.reshape(B, L, 1)
    idl_c = tree_ids_dl.reshape(B, L, 1)
    colid_spec = pl.BlockSpec((1, L, 1), lambda b: (b, 0, 0))
    bc_spec = pl.BlockSpec((1, L, 16), lambda b: (b, 0, 0))
    idd_bc, idr_bc, idl_bc = pl.pallas_call(
        functools.partial(_tc_prep_body, L=L),
        grid=(B,),
        in_specs=[colid_spec, colid_spec, colid_spec],
        out_specs=[bc_spec, bc_spec, bc_spec],
        out_shape=[jax.ShapeDtypeStruct((B, L, 16), jnp.int32)] * 3,
        compiler_params=pltpu.CompilerParams(
            dimension_semantics=("arbitrary",)),
    )(idd_c, idr_c, idl_c)

    sc_params = pltpu.CompilerParams(needs_layout_passes=False,
                                     use_tc_tiling_on_sc=False)
    sc1a = functools.partial(
        pl.kernel,
        out_type=[cvsd, cvsd],
        mesh=mesh,
        scratch_types=[
            pltpu.VMEM((L, 16), f32),
            pltpu.VMEM((L, 16), f32),
            pltpu.VMEM((L, 16), f32),
            pltpu.VMEM((8, CH2), jnp.int32),
            pltpu.VMEM((L, 16), jnp.int32),
            pltpu.VMEM((L, 16), jnp.int32),
            pltpu.SemaphoreType.DMA,
            pltpu.SemaphoreType.DMA,
        ],
        compiler_params=sc_params,
    )(functools.partial(_sc_scatter_body, B=B, L=L, H=H))
    sr, sl = sc1a(hx0_cv, idr_bc.reshape(BL, 16), idl_bc.reshape(BL, 16))

    sc1b = functools.partial(
        pl.kernel,
        out_type=[rowsd, rowsd, rowsd],
        mesh=mesh,
        scratch_types=[
            pltpu.VMEM((CH,), jnp.int32),
            pltpu.VMEM((CH,), jnp.int32),
            pltpu.VMEM((CH,), jnp.int32),
            pltpu.VMEM((CH, H), f32),
            pltpu.VMEM((CH, H), f32),
            pltpu.VMEM((CH, H), f32),
            pltpu.SemaphoreType.DMA,
            pltpu.SemaphoreType.DMA,
            pltpu.SemaphoreType.DMA,
        ],
        compiler_params=sc_params,
    )(functools.partial(_sc_gather_body, B=B, L=L, H=H))
    xg, hr, hl = sc1b(x2, hx0_2, idd_f, idr_f, idl_f)

    wh_spec = pl.BlockSpec((H, H), lambda b: (0, 0))
    bias_spec = pl.BlockSpec((1, H), lambda b: (0, 0))
    bh_spec = pl.BlockSpec((1, L, H), lambda b: (b, 0, 0))
    row_spec = pl.BlockSpec((1, 1, L), lambda b: (b, 0, 0))
    idr_r = tree_ids_dr.reshape(B, 1, L)
    idl_r = tree_ids_dl.reshape(B, 1, L)
    fc, iu, o = pl.pallas_call(
        functools.partial(_tc_gates_body, L=L, H=H),
        grid=(B,),
        in_specs=[
            pl.BlockSpec((1, L, E), lambda b: (b, 0, 0)),
            bh_spec, bh_spec, bh_spec, bh_spec, bh_spec, bh_spec,
            row_spec, row_spec,
            pl.BlockSpec((E, 3 * H), lambda b: (0, 0)),
            wh_spec, wh_spec,
            pl.BlockSpec((E, H), lambda b: (0, 0)),
            wh_spec, wh_spec,
            bias_spec, bias_spec, bias_spec,
        ],
        out_specs=[bh_spec, bh_spec, bh_spec],
        out_shape=[jax.ShapeDtypeStruct((B, L, H), f32)] * 3,
        compiler_params=pltpu.CompilerParams(
            dimension_semantics=("arbitrary",)),
    )(x, sr.reshape(B, L, H), sl.reshape(B, L, H), xg.reshape(B, L, H),
      hr.reshape(B, L, H), hl.reshape(B, L, H), hx_1, idr_r, idl_r,
      W_ioux, W_iouh0[:, :H], W_iouh1[:, :H], W_fx,
      W_fh0 + W_fh1, W_fh2 + W_fh3,
      b_iouh0[:H].reshape(1, H), b_iouh1[:H].reshape(1, H),
      (b_fh0 + b_fh1 + b_fh2 + b_fh3).reshape(1, H))

    sc2 = functools.partial(
        pl.kernel,
        out_type=[cvsd],
        mesh=mesh,
        scratch_types=[
            pltpu.VMEM((L, 16), f32),
            pltpu.VMEM((L, 16), f32),
            pltpu.VMEM((8, CH2), jnp.int32),
            pltpu.VMEM((L, 16), jnp.int32),
            pltpu.SemaphoreType.DMA,
        ],
        compiler_params=pltpu.CompilerParams(needs_layout_passes=False, use_tc_tiling_on_sc=False),
    )(functools.partial(_sc_scatter_fc_body, B=B, L=L, H=H))
    (cs,) = sc2(fc.reshape(BL * 16, 16), idd_bc.reshape(BL, 16))

    h_full, c_full = pl.pallas_call(
        _tc_combine_body,
        grid=(B,),
        in_specs=[bh_spec, bh_spec, bh_spec],
        out_specs=[bh_spec, bh_spec],
        out_shape=[jax.ShapeDtypeStruct((B, L, H), f32)] * 2,
        compiler_params=pltpu.CompilerParams(
            dimension_semantics=("arbitrary",)),
    )(iu, o, cs.reshape(B, L, H))

    idd_c = tree_ids_d.reshape(B, L, 1)
    col_spec = pl.BlockSpec((1, L, 1), lambda b: (b, 0, 0))
    h_out, c_out = pl.pallas_call(
        functools.partial(_stage_b_body, L=L, H=H, BL=BL),
        grid=(B,),
        in_specs=[
            pl.BlockSpec(memory_space=pl.ANY),
            pl.BlockSpec(memory_space=pl.ANY),
            bh_spec, bh_spec,
            col_spec,
        ],
        out_specs=[bh_spec, bh_spec],
        out_shape=[jax.ShapeDtypeStruct((B, L, H), f32)] * 2,
        scratch_shapes=[
            pltpu.VMEM((L + 16, H), f32),
            pltpu.VMEM((L + 16, H), f32),
            pltpu.SemaphoreType.DMA,
            pltpu.SemaphoreType.DMA,
            pltpu.SMEM((1,), jnp.int32),
        ],
        compiler_params=pltpu.CompilerParams(
            dimension_semantics=("arbitrary",)),
    )(h_full.reshape(BL, H), c_full.reshape(BL, H), hx_0, hx_1, idd_c)

    return (h_out, c_out)
